# Initial kernel scaffold; baseline (speedup 1.0000x reference)
import jax
import jax.numpy as jnp
from jax.experimental import pallas as pl


def kernel(x, edge_index, edge_attr, batch, conv1_nn_w1, conv1_nn_b1, conv1_nn_w2, conv1_nn_b2, conv1_root_w, conv1_bias, conv2_nn_w1, conv2_nn_b1, conv2_nn_w2, conv2_nn_b2, conv2_root_w, conv2_bias, conv3_nn_w1, conv3_nn_b1, conv3_nn_w2, conv3_nn_b2, conv3_root_w, conv3_bias, fc1_w, fc1_b, fc2_w, fc2_b, fc3_w, fc3_b):
    raise NotImplementedError("write your pallas kernel here")


# speedup vs baseline: 1.5136x; 1.5136x over previous
import functools

import jax
import jax.numpy as jnp
from jax.experimental import pallas as pl
from jax.experimental.pallas import tpu as pltpu

_SLOPE = 0.01
_KC = 8


def _ceil_to(a, b):
    return (a + b - 1) // b * b


def _leaky(v):
    return jnp.where(v >= 0, v, _SLOPE * v)


def _sigmoid(v):
    z = jnp.exp(-jnp.abs(v))
    return jnp.where(v >= 0, 1.0 / (1.0 + z), z / (1.0 + z))


def _params(dims):
    return pltpu.CompilerParams(dimension_semantics=dims,
                                vmem_limit_bytes=56 * 1024 * 1024)


def _msg_body(ea_ref, xs_ref, w1_ref, b1_ref, w2m_ref, w2t_ref, b2_ref,
              o_ref, h_sc, acc_sc, *, kc, out_ch, nk):
    k = pl.program_id(1)

    def chunk(h, w2c):
        xsk = xs_ref[0]
        part = jnp.dot(xsk, b2_ref[...],
                       preferred_element_type=jnp.float32)
        for kk in range(kc):
            w = jnp.dot(h, w2c[:, kk * out_ch:(kk + 1) * out_ch],
                        preferred_element_type=jnp.float32)
            part = part + xsk[:, kk:kk + 1] * w
        return part

    @pl.when(k == 0)
    def _():
        h = jnp.dot(ea_ref[...], w1_ref[...],
                    preferred_element_type=jnp.float32) + b1_ref[...]
        h = _leaky(h)
        h_sc[...] = h
        acc_sc[...] = chunk(h, w2t_ref[...])

    @pl.when(k > 0)
    def _():
        acc_sc[...] = acc_sc[...] + chunk(h_sc[...], w2m_ref[...])

    @pl.when(k == nk - 1)
    def _():
        o_ref[...] = acc_sc[...]


def _messages(ea8, xs_chunks, w2_main, w2_tail, b2_mat, w1p, b1, out_ch, te):
    nk, e_pad, _ = xs_chunks.shape
    hdim = w1p.shape[1]
    return pl.pallas_call(
        functools.partial(_msg_body, kc=_KC, out_ch=out_ch, nk=nk),
        out_shape=jax.ShapeDtypeStruct((e_pad, out_ch), jnp.float32),
        grid=(e_pad // te, nk),
        in_specs=[
            pl.BlockSpec((te, 8), lambda e, k: (e, 0)),
            pl.BlockSpec((1, te, _KC), lambda e, k: (k, e, 0)),
            pl.BlockSpec((8, hdim), lambda e, k: (0, 0)),
            pl.BlockSpec((1, hdim), lambda e, k: (0, 0)),
            pl.BlockSpec((hdim, _KC * out_ch),
                         lambda e, k: (0, jnp.maximum(k - 1, 0))),
            pl.BlockSpec((hdim, _KC * out_ch), lambda e, k: (0, 0)),
            pl.BlockSpec((_KC, out_ch), lambda e, k: (k, 0)),
        ],
        out_specs=pl.BlockSpec((te, out_ch), lambda e, k: (e, 0)),
        scratch_shapes=[pltpu.VMEM((te, hdim), jnp.float32),
                        pltpu.VMEM((te, out_ch), jnp.float32)],
        compiler_params=_params(("parallel", "arbitrary")),
    )(ea8, xs_chunks, w1p, b1, w2_main, w2_tail, b2_mat)


def _agg_body(tgt_ref, m_ref, x_ref, wr_ref, b_ref, o_ref, acc_sc, deg_sc,
              *, tn):
    n = pl.program_id(0)
    e = pl.program_id(1)

    @pl.when(e == 0)
    def _():
        acc_sc[...] = jnp.zeros_like(acc_sc)
        deg_sc[...] = jnp.zeros_like(deg_sc)

    tec = m_ref.shape[0]
    rows = (jax.lax.broadcasted_iota(jnp.int32, (tn, tec), 0)
            + n * tn).astype(jnp.float32)
    mask = (rows == tgt_ref[...]).astype(jnp.float32)
    acc_sc[...] += jnp.dot(mask, m_ref[...],
                           preferred_element_type=jnp.float32)
    deg_sc[...] += jnp.sum(mask, axis=1, keepdims=True)

    @pl.when(e == pl.num_programs(1) - 1)
    def _():
        root = jnp.dot(x_ref[...], wr_ref[...],
                       preferred_element_type=jnp.float32)
        inv = 1.0 / jnp.maximum(deg_sc[...], 1.0)
        o_ref[...] = _leaky(acc_sc[...] * inv + root + b_ref[...])


def _combine(tgtf, msgs, x_nodes, w_root, bias, tn, tec):
    n_pad = x_nodes.shape[0]
    e_pad = msgs.shape[0]
    out_ch = msgs.shape[1]
    kdim = x_nodes.shape[1]
    return pl.pallas_call(
        functools.partial(_agg_body, tn=tn),
        out_shape=jax.ShapeDtypeStruct((n_pad, out_ch), jnp.float32),
        grid=(n_pad // tn, e_pad // tec),
        in_specs=[
            pl.BlockSpec((1, tec), lambda n, e: (0, e)),
            pl.BlockSpec((tec, out_ch), lambda n, e: (e, 0)),
            pl.BlockSpec((tn, kdim), lambda n, e: (n, 0)),
            pl.BlockSpec((kdim, out_ch), lambda n, e: (0, 0)),
            pl.BlockSpec((1, out_ch), lambda n, e: (0, 0)),
        ],
        out_specs=pl.BlockSpec((tn, out_ch), lambda n, e: (n, 0)),
        scratch_shapes=[pltpu.VMEM((tn, out_ch), jnp.float32),
                        pltpu.VMEM((tn, 1), jnp.float32)],
        compiler_params=_params(("parallel", "arbitrary")),
    )(tgtf, msgs, x_nodes, w_root, bias)


def _head_body(bf_ref, d_ref, x_ref, w1d_ref, w1x_ref, b1_ref,
               w2_ref, b2_ref, w3_ref, b3_ref, o_ref,
               pd_sc, px_sc, cnt_sc, *, nb):
    n = pl.program_id(0)

    @pl.when(n == 0)
    def _():
        pd_sc[...] = jnp.zeros_like(pd_sc)
        px_sc[...] = jnp.zeros_like(px_sc)
        cnt_sc[...] = jnp.zeros_like(cnt_sc)

    tn = d_ref.shape[0]
    gids = jax.lax.broadcasted_iota(jnp.int32, (nb, tn), 0).astype(jnp.float32)
    mask = (gids == bf_ref[...]).astype(jnp.float32)
    pd_sc[...] += jnp.dot(mask, d_ref[...],
                          preferred_element_type=jnp.float32)
    px_sc[...] += jnp.dot(mask, x_ref[...],
                          preferred_element_type=jnp.float32)
    cnt_sc[...] += jnp.sum(mask, axis=1, keepdims=True)

    @pl.when(n == pl.num_programs(0) - 1)
    def _():
        inv = 1.0 / jnp.maximum(cnt_sc[...], 1.0)
        h = jnp.dot(pd_sc[...] * inv, w1d_ref[...],
                    preferred_element_type=jnp.float32) \
            + jnp.dot(px_sc[...] * inv, w1x_ref[...],
                      preferred_element_type=jnp.float32) + b1_ref[...]
        h = _leaky(h)
        h = _leaky(jnp.dot(h, w2_ref[...],
                           preferred_element_type=jnp.float32) + b2_ref[...])
        y = jnp.dot(h, w3_ref[...],
                    preferred_element_type=jnp.float32) + b3_ref[...]
        o_ref[...] = _sigmoid(y)


def _head(batchf, d3, x8, w1d, w1x, b1, w2, b2, w3, b3, nb, tn):
    n_pad, ddim = d3.shape
    h1 = w1d.shape[1]
    h2 = w2.shape[1]
    return pl.pallas_call(
        functools.partial(_head_body, nb=nb),
        out_shape=jax.ShapeDtypeStruct((nb, 1), jnp.float32),
        grid=(n_pad // tn,),
        in_specs=[
            pl.BlockSpec((1, tn), lambda n: (0, n)),
            pl.BlockSpec((tn, ddim), lambda n: (n, 0)),
            pl.BlockSpec((tn, 8), lambda n: (n, 0)),
            pl.BlockSpec((ddim, h1), lambda n: (0, 0)),
            pl.BlockSpec((8, h1), lambda n: (0, 0)),
            pl.BlockSpec((1, h1), lambda n: (0, 0)),
            pl.BlockSpec((h1, h2), lambda n: (0, 0)),
            pl.BlockSpec((1, h2), lambda n: (0, 0)),
            pl.BlockSpec((h2, 1), lambda n: (0, 0)),
            pl.BlockSpec((1, 1), lambda n: (0, 0)),
        ],
        out_specs=pl.BlockSpec((nb, 1), lambda n: (0, 0)),
        scratch_shapes=[pltpu.VMEM((nb, ddim), jnp.float32),
                        pltpu.VMEM((nb, 8), jnp.float32),
                        pltpu.VMEM((nb, 1), jnp.float32)],
        compiler_params=_params(("arbitrary",)),
    )(batchf, d3, x8, w1d, w1x, b1, w2, b2, w3, b3)


def _prep_conv(w2, b2, feats_src, out_ch, kdim, e_pad):
    hdim = w2.shape[0]
    nk = -(-kdim // _KC)
    k_main = (nk - 1) * _KC
    tail = kdim - k_main

    w2_tail = jnp.zeros((hdim, _KC * out_ch), jnp.float32)
    w2_tail = w2_tail.at[:, :tail * out_ch].set(w2[:, k_main * out_ch:])
    w2_main = w2[:, :k_main * out_ch] if k_main else w2_tail

    b2m = b2.reshape(kdim, out_ch)
    b2_mat = jnp.concatenate(
        [b2m[k_main:], jnp.zeros((_KC - tail, out_ch), jnp.float32),
         b2m[:k_main]], axis=0)

    e_have = feats_src.shape[0]
    xs = jnp.concatenate(
        [feats_src[:, k_main:], jnp.zeros((e_have, _KC - tail), jnp.float32),
         feats_src[:, :k_main]], axis=1)
    if e_pad > e_have:
        xs = jnp.concatenate(
            [xs, jnp.zeros((e_pad - e_have, nk * _KC), jnp.float32)], axis=0)
    xs_chunks = xs.reshape(e_pad, nk, _KC).transpose(1, 0, 2)
    return w2_main, w2_tail, b2_mat, xs_chunks


def kernel(x, edge_index, edge_attr, batch,
           conv1_nn_w1, conv1_nn_b1, conv1_nn_w2, conv1_nn_b2,
           conv1_root_w, conv1_bias,
           conv2_nn_w1, conv2_nn_b1, conv2_nn_w2, conv2_nn_b2,
           conv2_root_w, conv2_bias,
           conv3_nn_w1, conv3_nn_b1, conv3_nn_w2, conv3_nn_b2,
           conv3_root_w, conv3_bias,
           fc1_w, fc1_b, fc2_w, fc2_b, fc3_w, fc3_b):
    num_graphs = 64
    x = x.astype(jnp.float32)
    n_nodes, fdim = x.shape
    n_edges = edge_index.shape[1]

    te = 2048 if n_edges % 2048 == 0 else _ceil_to(n_edges, 128)
    e_pad = _ceil_to(n_edges, te)
    tec = te
    n_pad = _ceil_to(n_nodes, 8)
    tn = 512 if n_pad % 512 == 0 else n_pad
    nb = _ceil_to(num_graphs, 8)

    src = edge_index[0]
    tgtf = jnp.full((1, e_pad), -1.0, jnp.float32).at[0, :n_edges].set(
        edge_index[1].astype(jnp.float32))
    batchf = jnp.full((1, n_pad), -1.0, jnp.float32).at[0, :n_nodes].set(
        batch.astype(jnp.float32))

    ea8 = jnp.zeros((e_pad, 8), jnp.float32).at[:n_edges, :4].set(
        edge_attr.astype(jnp.float32))
    x_pad = jnp.zeros((n_pad, fdim), jnp.float32).at[:n_nodes].set(x)
    x8 = jnp.zeros((n_pad, 8), jnp.float32).at[:n_nodes, :4].set(x)

    def conv(d_nodes, w1, b1, w2, b2, w_root, bias, out_ch):
        kdim = d_nodes.shape[1]
        w2_main, w2_tail, b2_mat, xs_chunks = _prep_conv(
            w2.astype(jnp.float32), b2.astype(jnp.float32),
            d_nodes[src], out_ch, kdim, e_pad)
        w1p = jnp.zeros((8, w1.shape[1]), jnp.float32).at[:w1.shape[0]].set(
            w1.astype(jnp.float32))
        msgs = _messages(ea8, xs_chunks, w2_main, w2_tail, b2_mat,
                         w1p, b1.reshape(1, -1).astype(jnp.float32),
                         out_ch, te)
        return _combine(tgtf, msgs, d_nodes,
                        w_root.astype(jnp.float32),
                        bias.reshape(1, -1).astype(jnp.float32), tn, tec)

    c1 = conv(x_pad, conv1_nn_w1, conv1_nn_b1, conv1_nn_w2, conv1_nn_b2,
              conv1_root_w, conv1_bias, 256)
    d1 = jnp.concatenate([c1, x_pad], axis=1)
    c2 = conv(d1, conv2_nn_w1, conv2_nn_b1, conv2_nn_w2, conv2_nn_b2,
              conv2_root_w, conv2_bias, 256)
    d2 = jnp.concatenate([c2, x_pad], axis=1)
    c3 = conv(d2, conv3_nn_w1, conv3_nn_b1, conv3_nn_w2, conv3_nn_b2,
              conv3_root_w, conv3_bias, 512)

    ddim = c3.shape[1]
    w1d = fc1_w[:ddim].astype(jnp.float32)
    w1x = jnp.zeros((8, fc1_w.shape[1]), jnp.float32).at[:fdim].set(
        fc1_w[ddim:].astype(jnp.float32))
    out = _head(batchf, c3, x8, w1d, w1x,
                fc1_b.reshape(1, -1).astype(jnp.float32),
                fc2_w.astype(jnp.float32),
                fc2_b.reshape(1, -1).astype(jnp.float32),
                fc3_w.astype(jnp.float32),
                fc3_b.reshape(1, -1).astype(jnp.float32), nb, tn)
    return out[:num_graphs]
